# Initial kernel scaffold; baseline (speedup 1.0000x reference)
import functools

import jax
import jax.numpy as jnp
from jax import lax
from jax.experimental import pallas as pl
from jax.experimental.pallas import tpu as pltpu
from jax.experimental.pallas import tpu_sc as plsc

N = 10000
E = 320000
D = 128
BGRAPH = 64
EPS = 1e-5
NEG = 0.2
W_COLS = 144

NC, NS = 2, 16
NW = NC * NS
KE = 64
NCHUNK = E // KE
ROWS_PER_TILE = N // NS

RB = 1000
NBLK = N // RB


def _prep_body(x_ref, wl_ref, bl_ref, wr_ref, br_ref, att_ref,
               xl_ref, xr_ref, sr_ref):
    xb = x_ref[...]
    cdims = (((1,), (1,)), ((), ()))
    xl = lax.dot_general(xb, wl_ref[...], cdims,
                         preferred_element_type=jnp.float32) + bl_ref[...]
    xr = lax.dot_general(xb, wr_ref[...], cdims,
                         preferred_element_type=jnp.float32) + br_ref[...]
    v = xl + xr
    m = jnp.maximum(v, NEG * v)
    ea = jnp.exp(jnp.sum(m * att_ref[...], axis=1, keepdims=True))
    xl_ref[...] = xl
    xr_ref[...] = xr
    sr_ref[...] = jnp.concatenate(
        [ea * xl, ea, jnp.zeros((RB, W_COLS - D - 1), jnp.float32)], axis=1)


_prep_call = pl.pallas_call(
    _prep_body,
    grid=(NBLK,),
    in_specs=[
        pl.BlockSpec((RB, D), lambda i: (i, 0)),
        pl.BlockSpec((D, D), lambda i: (0, 0)),
        pl.BlockSpec((1, D), lambda i: (0, 0)),
        pl.BlockSpec((D, D), lambda i: (0, 0)),
        pl.BlockSpec((1, D), lambda i: (0, 0)),
        pl.BlockSpec((1, D), lambda i: (0, 0)),
    ],
    out_specs=[
        pl.BlockSpec((RB, D), lambda i: (i, 0)),
        pl.BlockSpec((RB, D), lambda i: (i, 0)),
        pl.BlockSpec((RB, W_COLS), lambda i: (i, 0)),
    ],
    out_shape=[
        jax.ShapeDtypeStruct((N, D), jnp.float32),
        jax.ShapeDtypeStruct((N, D), jnp.float32),
        jax.ShapeDtypeStruct((N, W_COLS), jnp.float32),
    ],
)


_sc_mesh = plsc.VectorSubcoreMesh(core_axis_name="c", subcore_axis_name="s",
                                  num_cores=NC, num_subcores=NS)


@functools.partial(
    pl.kernel,
    out_type=jax.ShapeDtypeStruct((NC, N, W_COLS), jnp.float32),
    mesh=_sc_mesh,
    scratch_types=[
        pltpu.VMEM((KE,), jnp.int32),
        pltpu.VMEM((KE,), jnp.int32),
        pltpu.VMEM((KE, D), jnp.float32),
        pltpu.VMEM((KE, D), jnp.float32),
        pltpu.VMEM((KE, W_COLS), jnp.float32),
        pltpu.VMEM((D,), jnp.float32),
        pltpu.VMEM_SHARED((N, W_COLS), jnp.float32),
        pltpu.SemaphoreType.DMA,
        pltpu.SemaphoreType.DMA,
    ],
)
def _sc_edge(xl_hbm, xr_hbm, src_hbm, dst_hbm, att_hbm, zeros_hbm, out_hbm,
             idx_s, idx_d, rl, rr, pay, att_v, acc, sem_a, sem_b):
    cid = lax.axis_index("c")
    sid = lax.axis_index("s")
    wid = sid * NC + cid

    r0 = sid * ROWS_PER_TILE
    pltpu.sync_copy(zeros_hbm.at[pl.ds(r0, ROWS_PER_TILE)],
                    acc.at[pl.ds(r0, ROWS_PER_TILE)])
    pltpu.sync_copy(att_hbm, att_v)
    plsc.subcore_barrier()

    att_sl = [att_v[pl.ds(16 * k, 16)] for k in range(D // 16)]
    lane = lax.iota(jnp.int32, 16)

    nfull = NCHUNK // NW
    extra = NCHUNK % NW
    my_n = nfull + jnp.where(wid < extra, 1, 0).astype(jnp.int32)

    def chunk_body(t, carry):
        base = (wid + t * NW) * KE
        pltpu.sync_copy(src_hbm.at[pl.ds(base, KE)], idx_s)
        pltpu.sync_copy(dst_hbm.at[pl.ds(base, KE)], idx_d)
        cp_a = pltpu.async_copy(xl_hbm.at[idx_s], rl, sem_a)
        cp_b = pltpu.async_copy(xr_hbm.at[idx_d], rr, sem_b)
        cp_a.wait()
        cp_b.wait()
        for e in range(KE):
            rls = [rl[e, pl.ds(16 * k, 16)] for k in range(D // 16)]
            accv = None
            for k in range(D // 16):
                v = rls[k] + rr[e, pl.ds(16 * k, 16)]
                m = jnp.maximum(v, NEG * v)
                term = m * att_sl[k]
                accv = term if accv is None else accv + term
            a = jnp.sum(accv)
            ea = jnp.exp(lax.broadcast_in_dim(a, (16,), ()))
            for k in range(D // 16):
                pay[e, pl.ds(16 * k, 16)] = ea * rls[k]
            pay[e, pl.ds(D, 16)] = jnp.where(lane == 0, ea, 0.0)
        pltpu.sync_copy(pay, acc.at[idx_d], add=True)
        return carry

    lax.fori_loop(0, my_n, chunk_body, 0)
    plsc.subcore_barrier()
    pltpu.sync_copy(acc.at[pl.ds(r0, ROWS_PER_TILE)],
                    out_hbm.at[cid, pl.ds(r0, ROWS_PER_TILE)])


def _combine_body(p_ref, ps_ref, bias_ref, h_ref, st_ref):
    s = p_ref[0] + p_ref[1] + ps_ref[...]
    h = s[:, :D] / (s[:, D:D + 1] + 1e-16) + bias_ref[...]
    h_ref[...] = h

    @pl.when(pl.program_id(0) == 0)
    def _():
        st_ref[...] = jnp.zeros_like(st_ref)

    st_ref[...] += jnp.concatenate(
        [jnp.sum(h, axis=0, keepdims=True),
         jnp.sum(h * h, axis=0, keepdims=True),
         jnp.zeros((6, D), jnp.float32)], axis=0)


_combine_call = pl.pallas_call(
    _combine_body,
    grid=(NBLK,),
    in_specs=[
        pl.BlockSpec((NC, RB, W_COLS), lambda i: (0, i, 0)),
        pl.BlockSpec((RB, W_COLS), lambda i: (i, 0)),
        pl.BlockSpec((1, D), lambda i: (0, 0)),
    ],
    out_specs=[
        pl.BlockSpec((RB, D), lambda i: (i, 0)),
        pl.BlockSpec((8, D), lambda i: (0, 0)),
    ],
    out_shape=[
        jax.ShapeDtypeStruct((N, D), jnp.float32),
        jax.ShapeDtypeStruct((8, D), jnp.float32),
    ],
)


def _norm_prep_body(h_ref, st_ref, gnw_ref, gnb_ref, gnms_ref, av_ref,
                    wl_ref, bl_ref, wr_ref, br_ref, att_ref,
                    xl_ref, xr_ref, sr_ref):
    st = st_ref[...]
    mean = st[0:1, :] * (1.0 / N)
    ex2 = st[1:2, :] * (1.0 / N)
    ms = gnms_ref[...]
    var = ex2 - (2.0 * ms - ms * ms) * (mean * mean)
    inv = lax.rsqrt(var + EPS)
    hn = gnw_ref[...] * (h_ref[...] - ms * mean) * inv + gnb_ref[...]
    h = jnp.where(hn >= 0, hn, av_ref[...] * hn)

    cdims = (((1,), (1,)), ((), ()))
    xl = lax.dot_general(h, wl_ref[...], cdims,
                         preferred_element_type=jnp.float32) + bl_ref[...]
    xr = lax.dot_general(h, wr_ref[...], cdims,
                         preferred_element_type=jnp.float32) + br_ref[...]
    v = xl + xr
    m = jnp.maximum(v, NEG * v)
    ea = jnp.exp(jnp.sum(m * att_ref[...], axis=1, keepdims=True))
    xl_ref[...] = xl
    xr_ref[...] = xr
    sr_ref[...] = jnp.concatenate(
        [ea * xl, ea, jnp.zeros((RB, W_COLS - D - 1), jnp.float32)], axis=1)


_norm_prep_call = pl.pallas_call(
    _norm_prep_body,
    grid=(NBLK,),
    in_specs=[
        pl.BlockSpec((RB, D), lambda i: (i, 0)),
        pl.BlockSpec((8, D), lambda i: (0, 0)),
        pl.BlockSpec((1, D), lambda i: (0, 0)),
        pl.BlockSpec((1, D), lambda i: (0, 0)),
        pl.BlockSpec((1, D), lambda i: (0, 0)),
        pl.BlockSpec((1, D), lambda i: (0, 0)),
        pl.BlockSpec((D, D), lambda i: (0, 0)),
        pl.BlockSpec((1, D), lambda i: (0, 0)),
        pl.BlockSpec((D, D), lambda i: (0, 0)),
        pl.BlockSpec((1, D), lambda i: (0, 0)),
        pl.BlockSpec((1, D), lambda i: (0, 0)),
    ],
    out_specs=[
        pl.BlockSpec((RB, D), lambda i: (i, 0)),
        pl.BlockSpec((RB, D), lambda i: (i, 0)),
        pl.BlockSpec((RB, W_COLS), lambda i: (i, 0)),
    ],
    out_shape=[
        jax.ShapeDtypeStruct((N, D), jnp.float32),
        jax.ShapeDtypeStruct((N, D), jnp.float32),
        jax.ShapeDtypeStruct((N, W_COLS), jnp.float32),
    ],
)


def _norm_pool_body(h_ref, st_ref, gnw_ref, gnb_ref, gnms_ref, av_ref,
                    bcol_ref, pool_ref):
    st = st_ref[...]
    mean = st[0:1, :] * (1.0 / N)
    ex2 = st[1:2, :] * (1.0 / N)
    ms = gnms_ref[...]
    var = ex2 - (2.0 * ms - ms * ms) * (mean * mean)
    inv = lax.rsqrt(var + EPS)
    hn = gnw_ref[...] * (h_ref[...] - ms * mean) * inv + gnb_ref[...]
    h = jnp.where(hn >= 0, hn, av_ref[...] * hn)

    onehot = (bcol_ref[...] ==
              lax.broadcasted_iota(jnp.int32, (1, BGRAPH), 1)
              ).astype(jnp.float32)
    part = lax.dot_general(onehot, h, (((0,), (0,)), ((), ())),
                           preferred_element_type=jnp.float32)

    @pl.when(pl.program_id(0) == 0)
    def _():
        pool_ref[...] = jnp.zeros_like(pool_ref)

    pool_ref[...] += part


_norm_pool_call = pl.pallas_call(
    _norm_pool_body,
    grid=(NBLK,),
    in_specs=[
        pl.BlockSpec((RB, D), lambda i: (i, 0)),
        pl.BlockSpec((8, D), lambda i: (0, 0)),
        pl.BlockSpec((1, D), lambda i: (0, 0)),
        pl.BlockSpec((1, D), lambda i: (0, 0)),
        pl.BlockSpec((1, D), lambda i: (0, 0)),
        pl.BlockSpec((1, D), lambda i: (0, 0)),
        pl.BlockSpec((RB, 1), lambda i: (i, 0)),
    ],
    out_specs=pl.BlockSpec((BGRAPH, D), lambda i: (0, 0)),
    out_shape=jax.ShapeDtypeStruct((BGRAPH, D), jnp.float32),
)


def _final_body(pool_ref, gnw_ref, gnb_ref, gnms_ref, wf_ref, bf_ref, z_ref):
    p = pool_ref[...]
    mean = jnp.sum(p, axis=0, keepdims=True) * (1.0 / BGRAPH)
    ms = gnms_ref[...]
    xc = p - ms * mean
    var = jnp.sum(xc * xc, axis=0, keepdims=True) * (1.0 / BGRAPH)
    pn = gnw_ref[...] * xc * lax.rsqrt(var + EPS) + gnb_ref[...]
    z = lax.dot_general(pn, wf_ref[...], (((1,), (1,)), ((), ())),
                        preferred_element_type=jnp.float32) + bf_ref[...]
    z_ref[...] = z


_final_call = pl.pallas_call(
    _final_body,
    out_shape=jax.ShapeDtypeStruct((BGRAPH, BGRAPH), jnp.float32),
)


def kernel(x, edge_index, batch, Wl0, bl0, Wr0, br0, att0, bias0,
           Wl1, bl1, Wr1, br1, att1, bias1, a0, a1,
           gn_w, gn_b, gn_ms, Wf, bf):
    src = edge_index[0]
    dst = edge_index[1]
    zeros_w = jnp.zeros((N, W_COLS), jnp.float32)
    row = lambda v: v.reshape(1, -1)
    a0v = jnp.broadcast_to(a0.reshape(1, 1), (1, D))
    a1v = jnp.broadcast_to(a1.reshape(1, 1), (1, D))
    bcol = batch.reshape(N, 1)

    xl0, xr0, sr0 = _prep_call(x, Wl0, row(bl0), Wr0, row(br0),
                               att0.reshape(1, D))
    parts0 = _sc_edge(xl0, xr0, src, dst, att0.reshape(D), zeros_w)
    h0, st0 = _combine_call(parts0, sr0, row(bias0))

    xl1, xr1, sr1 = _norm_prep_call(h0, st0, row(gn_w), row(gn_b),
                                    row(gn_ms), a0v, Wl1, row(bl1),
                                    Wr1, row(br1), att1.reshape(1, D))
    parts1 = _sc_edge(xl1, xr1, src, dst, att1.reshape(D), zeros_w)
    h1, st1 = _combine_call(parts1, sr1, row(bias1))

    pooled = _norm_pool_call(h1, st1, row(gn_w), row(gn_b), row(gn_ms),
                             a1v, bcol)

    z = _final_call(pooled, row(gn_w), row(gn_b), row(gn_ms), Wf, row(bf))
    return z


# speedup vs baseline: 9.4812x; 9.4812x over previous
import functools

import jax
import jax.numpy as jnp
from jax import lax
from jax.experimental import pallas as pl
from jax.experimental.pallas import tpu as pltpu
from jax.experimental.pallas import tpu_sc as plsc

N = 10000
E = 320000
D = 128
BGRAPH = 64
EPS = 1e-5
NEG = 0.2
W_COLS = 144

NC, NS = 2, 16
NW = NC * NS
KE = 64
NCHUNK = E // KE
NPAD = 10240
ROWS_PER_TILE = NPAD // NS

RB = 1000
NBLK = N // RB


def _prep_body(x_ref, wl_ref, bl_ref, wr_ref, br_ref, att_ref,
               xl_ref, xr_ref, sr_ref):
    xb = x_ref[...]
    cdims = (((1,), (1,)), ((), ()))
    xl = lax.dot_general(xb, wl_ref[...], cdims,
                         preferred_element_type=jnp.float32) + bl_ref[...]
    xr = lax.dot_general(xb, wr_ref[...], cdims,
                         preferred_element_type=jnp.float32) + br_ref[...]
    v = xl + xr
    m = jnp.maximum(v, NEG * v)
    ea = jnp.exp(jnp.sum(m * att_ref[...], axis=1, keepdims=True))
    xl_ref[...] = xl
    xr_ref[...] = xr
    sr_ref[...] = jnp.concatenate(
        [ea * xl, ea, jnp.zeros((RB, W_COLS - D - 1), jnp.float32)], axis=1)


_prep_call = pl.pallas_call(
    _prep_body,
    grid=(NBLK,),
    in_specs=[
        pl.BlockSpec((RB, D), lambda i: (i, 0)),
        pl.BlockSpec((D, D), lambda i: (0, 0)),
        pl.BlockSpec((1, D), lambda i: (0, 0)),
        pl.BlockSpec((D, D), lambda i: (0, 0)),
        pl.BlockSpec((1, D), lambda i: (0, 0)),
        pl.BlockSpec((1, D), lambda i: (0, 0)),
    ],
    out_specs=[
        pl.BlockSpec((RB, D), lambda i: (i, 0)),
        pl.BlockSpec((RB, D), lambda i: (i, 0)),
        pl.BlockSpec((RB, W_COLS), lambda i: (i, 0)),
    ],
    out_shape=[
        jax.ShapeDtypeStruct((N, D), jnp.float32),
        jax.ShapeDtypeStruct((N, D), jnp.float32),
        jax.ShapeDtypeStruct((N, W_COLS), jnp.float32),
    ],
)


_sc_mesh = plsc.VectorSubcoreMesh(core_axis_name="c", subcore_axis_name="s",
                                  num_cores=NC, num_subcores=NS)


@functools.partial(
    pl.kernel,
    out_type=[
        jax.ShapeDtypeStruct((NC, NPAD, D), jnp.float32),
        jax.ShapeDtypeStruct((NC, NPAD), jnp.float32),
    ],
    mesh=_sc_mesh,
    scratch_types=[
        pltpu.VMEM((KE,), jnp.int32),
        pltpu.VMEM((KE,), jnp.int32),
        pltpu.VMEM((KE, D), jnp.float32),
        pltpu.VMEM((KE, D), jnp.float32),
        pltpu.VMEM((KE, D), jnp.float32),
        pltpu.VMEM((KE,), jnp.float32),
        pltpu.VMEM((D,), jnp.float32),
        pltpu.VMEM((ROWS_PER_TILE,), jnp.float32),
        pltpu.VMEM_SHARED((NPAD, D), jnp.float32),
        pltpu.VMEM_SHARED((NPAD,), jnp.float32),
        pltpu.SemaphoreType.DMA,
        pltpu.SemaphoreType.DMA,
    ],
)
def _sc_edge(xl_hbm, xr_hbm, src_hbm, dst_hbm, att_hbm, zn_hbm, zd_hbm,
             outn_hbm, outd_hbm,
             idx_s, idx_d, rl, rr, pay, dpay, att_v, stage,
             acc_n, acc_d, sem_a, sem_b):
    cid = lax.axis_index("c")
    sid = lax.axis_index("s")
    wid = sid * NC + cid

    r0 = sid * ROWS_PER_TILE
    pltpu.sync_copy(zn_hbm.at[pl.ds(r0, ROWS_PER_TILE)],
                    acc_n.at[pl.ds(r0, ROWS_PER_TILE)])
    pltpu.sync_copy(zd_hbm.at[pl.ds(r0, ROWS_PER_TILE)], stage)
    pltpu.sync_copy(stage, acc_d.at[pl.ds(r0, ROWS_PER_TILE)])
    pltpu.sync_copy(att_hbm, att_v)
    plsc.subcore_barrier()

    att_sl = [att_v[pl.ds(16 * k, 16)] for k in range(D // 16)]
    lane = lax.iota(jnp.int32, 16)
    bidx = [jnp.bitwise_xor(lane, k) for k in (8, 4, 2, 1)]

    nfull = NCHUNK // NW
    extra = NCHUNK % NW
    my_n = nfull + jnp.where(wid < extra, 1, 0).astype(jnp.int32)

    def chunk_body(t, carry):
        base = (wid + t * NW) * KE
        pltpu.sync_copy(src_hbm.at[pl.ds(base, KE)], idx_s)
        pltpu.sync_copy(dst_hbm.at[pl.ds(base, KE)], idx_d)
        cp_a = pltpu.async_copy(xl_hbm.at[idx_s], rl, sem_a)
        cp_b = pltpu.async_copy(xr_hbm.at[idx_d], rr, sem_b)
        cp_a.wait()
        cp_b.wait()
        for g in range(KE // 16):
            dacc = jnp.zeros((16,), jnp.float32)
            for j in range(16):
                e = 16 * g + j
                rls = [rl[e, pl.ds(16 * k, 16)] for k in range(D // 16)]
                accv = None
                for k in range(D // 16):
                    v = rls[k] + rr[e, pl.ds(16 * k, 16)]
                    m = jnp.maximum(v, NEG * v)
                    term = m * att_sl[k]
                    accv = term if accv is None else accv + term
                for bx in bidx:
                    accv = accv + accv[bx]
                ea = jnp.exp(accv)
                for k in range(D // 16):
                    pay[e, pl.ds(16 * k, 16)] = ea * rls[k]
                dacc = jnp.where(lane == j, ea, dacc)
            dpay[pl.ds(16 * g, 16)] = dacc
        pltpu.sync_copy(pay, acc_n.at[idx_d], add=True)
        pltpu.sync_copy(dpay, acc_d.at[idx_d], add=True)
        return carry

    lax.fori_loop(0, my_n, chunk_body, 0)
    plsc.subcore_barrier()
    pltpu.sync_copy(acc_n.at[pl.ds(r0, ROWS_PER_TILE)],
                    outn_hbm.at[cid, pl.ds(r0, ROWS_PER_TILE)])
    pltpu.sync_copy(acc_d.at[pl.ds(r0, ROWS_PER_TILE)], stage)
    pltpu.sync_copy(stage, outd_hbm.at[cid, pl.ds(r0, ROWS_PER_TILE)])


def _combine_body(pn_ref, pd_ref, ps_ref, bias_ref, h_ref, st_ref):
    ps = ps_ref[...]
    num = pn_ref[0] + pn_ref[1] + ps[:, :D]
    dcol = lax.dot_general(pd_ref[0], jnp.ones((NC, 1), jnp.float32),
                           (((0,), (0,)), ((), ())),
                           preferred_element_type=jnp.float32)
    den = dcol + ps[:, D:D + 1]
    h = num / (den + 1e-16) + bias_ref[...]
    h_ref[...] = h

    @pl.when(pl.program_id(0) == 0)
    def _():
        st_ref[...] = jnp.zeros_like(st_ref)

    st_ref[...] += jnp.concatenate(
        [jnp.sum(h, axis=0, keepdims=True),
         jnp.sum(h * h, axis=0, keepdims=True),
         jnp.zeros((6, D), jnp.float32)], axis=0)


_combine_call = pl.pallas_call(
    _combine_body,
    grid=(NBLK,),
    in_specs=[
        pl.BlockSpec((NC, RB, D), lambda i: (0, i, 0)),
        pl.BlockSpec((1, NC, RB), lambda i: (i, 0, 0)),
        pl.BlockSpec((RB, W_COLS), lambda i: (i, 0)),
        pl.BlockSpec((1, D), lambda i: (0, 0)),
    ],
    out_specs=[
        pl.BlockSpec((RB, D), lambda i: (i, 0)),
        pl.BlockSpec((8, D), lambda i: (0, 0)),
    ],
    out_shape=[
        jax.ShapeDtypeStruct((N, D), jnp.float32),
        jax.ShapeDtypeStruct((8, D), jnp.float32),
    ],
)


def _norm_prep_body(h_ref, st_ref, gnw_ref, gnb_ref, gnms_ref, av_ref,
                    wl_ref, bl_ref, wr_ref, br_ref, att_ref,
                    xl_ref, xr_ref, sr_ref):
    st = st_ref[...]
    mean = st[0:1, :] * (1.0 / N)
    ex2 = st[1:2, :] * (1.0 / N)
    ms = gnms_ref[...]
    var = ex2 - (2.0 * ms - ms * ms) * (mean * mean)
    inv = lax.rsqrt(var + EPS)
    hn = gnw_ref[...] * (h_ref[...] - ms * mean) * inv + gnb_ref[...]
    h = jnp.where(hn >= 0, hn, av_ref[...] * hn)

    cdims = (((1,), (1,)), ((), ()))
    xl = lax.dot_general(h, wl_ref[...], cdims,
                         preferred_element_type=jnp.float32) + bl_ref[...]
    xr = lax.dot_general(h, wr_ref[...], cdims,
                         preferred_element_type=jnp.float32) + br_ref[...]
    v = xl + xr
    m = jnp.maximum(v, NEG * v)
    ea = jnp.exp(jnp.sum(m * att_ref[...], axis=1, keepdims=True))
    xl_ref[...] = xl
    xr_ref[...] = xr
    sr_ref[...] = jnp.concatenate(
        [ea * xl, ea, jnp.zeros((RB, W_COLS - D - 1), jnp.float32)], axis=1)


_norm_prep_call = pl.pallas_call(
    _norm_prep_body,
    grid=(NBLK,),
    in_specs=[
        pl.BlockSpec((RB, D), lambda i: (i, 0)),
        pl.BlockSpec((8, D), lambda i: (0, 0)),
        pl.BlockSpec((1, D), lambda i: (0, 0)),
        pl.BlockSpec((1, D), lambda i: (0, 0)),
        pl.BlockSpec((1, D), lambda i: (0, 0)),
        pl.BlockSpec((1, D), lambda i: (0, 0)),
        pl.BlockSpec((D, D), lambda i: (0, 0)),
        pl.BlockSpec((1, D), lambda i: (0, 0)),
        pl.BlockSpec((D, D), lambda i: (0, 0)),
        pl.BlockSpec((1, D), lambda i: (0, 0)),
        pl.BlockSpec((1, D), lambda i: (0, 0)),
    ],
    out_specs=[
        pl.BlockSpec((RB, D), lambda i: (i, 0)),
        pl.BlockSpec((RB, D), lambda i: (i, 0)),
        pl.BlockSpec((RB, W_COLS), lambda i: (i, 0)),
    ],
    out_shape=[
        jax.ShapeDtypeStruct((N, D), jnp.float32),
        jax.ShapeDtypeStruct((N, D), jnp.float32),
        jax.ShapeDtypeStruct((N, W_COLS), jnp.float32),
    ],
)


def _norm_pool_body(h_ref, st_ref, gnw_ref, gnb_ref, gnms_ref, av_ref,
                    bcol_ref, pool_ref):
    st = st_ref[...]
    mean = st[0:1, :] * (1.0 / N)
    ex2 = st[1:2, :] * (1.0 / N)
    ms = gnms_ref[...]
    var = ex2 - (2.0 * ms - ms * ms) * (mean * mean)
    inv = lax.rsqrt(var + EPS)
    hn = gnw_ref[...] * (h_ref[...] - ms * mean) * inv + gnb_ref[...]
    h = jnp.where(hn >= 0, hn, av_ref[...] * hn)

    onehot = (bcol_ref[...] ==
              lax.broadcasted_iota(jnp.int32, (1, BGRAPH), 1)
              ).astype(jnp.float32)
    part = lax.dot_general(onehot, h, (((0,), (0,)), ((), ())),
                           preferred_element_type=jnp.float32)

    @pl.when(pl.program_id(0) == 0)
    def _():
        pool_ref[...] = jnp.zeros_like(pool_ref)

    pool_ref[...] += part


_norm_pool_call = pl.pallas_call(
    _norm_pool_body,
    grid=(NBLK,),
    in_specs=[
        pl.BlockSpec((RB, D), lambda i: (i, 0)),
        pl.BlockSpec((8, D), lambda i: (0, 0)),
        pl.BlockSpec((1, D), lambda i: (0, 0)),
        pl.BlockSpec((1, D), lambda i: (0, 0)),
        pl.BlockSpec((1, D), lambda i: (0, 0)),
        pl.BlockSpec((1, D), lambda i: (0, 0)),
        pl.BlockSpec((RB, 1), lambda i: (i, 0)),
    ],
    out_specs=pl.BlockSpec((BGRAPH, D), lambda i: (0, 0)),
    out_shape=jax.ShapeDtypeStruct((BGRAPH, D), jnp.float32),
)


def _final_body(pool_ref, gnw_ref, gnb_ref, gnms_ref, wf_ref, bf_ref, z_ref):
    p = pool_ref[...]
    mean = jnp.sum(p, axis=0, keepdims=True) * (1.0 / BGRAPH)
    ms = gnms_ref[...]
    xc = p - ms * mean
    var = jnp.sum(xc * xc, axis=0, keepdims=True) * (1.0 / BGRAPH)
    pn = gnw_ref[...] * xc * lax.rsqrt(var + EPS) + gnb_ref[...]
    z = lax.dot_general(pn, wf_ref[...], (((1,), (1,)), ((), ())),
                        preferred_element_type=jnp.float32) + bf_ref[...]
    z_ref[...] = z


_final_call = pl.pallas_call(
    _final_body,
    out_shape=jax.ShapeDtypeStruct((BGRAPH, BGRAPH), jnp.float32),
)


def kernel(x, edge_index, batch, Wl0, bl0, Wr0, br0, att0, bias0,
           Wl1, bl1, Wr1, br1, att1, bias1, a0, a1,
           gn_w, gn_b, gn_ms, Wf, bf):
    src = edge_index[0]
    dst = edge_index[1]
    zeros_n = jnp.zeros((NPAD, D), jnp.float32)
    zeros_d = jnp.zeros((NPAD,), jnp.float32)
    row = lambda v: v.reshape(1, -1)
    a0v = jnp.broadcast_to(a0.reshape(1, 1), (1, D))
    a1v = jnp.broadcast_to(a1.reshape(1, 1), (1, D))
    bcol = batch.reshape(N, 1)

    xl0, xr0, sr0 = _prep_call(x, Wl0, row(bl0), Wr0, row(br0),
                               att0.reshape(1, D))
    pn0, pd0 = _sc_edge(xl0, xr0, src, dst, att0.reshape(D),
                        zeros_n, zeros_d)
    pd0r = pd0[:, :N].reshape(NC, NBLK, RB).transpose(1, 0, 2)
    h0, st0 = _combine_call(pn0, pd0r, sr0, row(bias0))

    xl1, xr1, sr1 = _norm_prep_call(h0, st0, row(gn_w), row(gn_b),
                                    row(gn_ms), a0v, Wl1, row(bl1),
                                    Wr1, row(br1), att1.reshape(1, D))
    pn1, pd1 = _sc_edge(xl1, xr1, src, dst, att1.reshape(D),
                        zeros_n, zeros_d)
    pd1r = pd1[:, :N].reshape(NC, NBLK, RB).transpose(1, 0, 2)
    h1, st1 = _combine_call(pn1, pd1r, sr1, row(bias1))

    pooled = _norm_pool_call(h1, st1, row(gn_w), row(gn_b), row(gn_ms),
                             a1v, bcol)

    z = _final_call(pooled, row(gn_w), row(gn_b), row(gn_ms), Wf, row(bf))
    return z
